# initial kernel scaffold (unmeasured)
import jax
import jax.numpy as jnp
from jax import lax
from jax.experimental import pallas as pl
from jax.experimental.pallas import tpu as pltpu
import functools

N_DEV = 4
M = 3072
N = 3072
CH = M // N_DEV


def kernel(A, B):
    A16 = A.astype(jnp.bfloat16)
    B16 = B.astype(jnp.bfloat16)

    def body(a_ref, b_ref, out_ref, recv_rs, rs_send, rs_recv, ag_send, ag_recv):
        my = lax.axis_index("i")
        left = (my - 1) % N_DEV
        right = (my + 1) % N_DEV

        def partial_chunk(c):
            a = a_ref[pl.ds(c * CH, CH), :]
            return jnp.dot(
                a, b_ref[:, :], preferred_element_type=jnp.float32
            ).astype(jnp.bfloat16)

        barrier_sem = pltpu.get_barrier_semaphore()
        for nbr in [left, right]:
            pl.semaphore_signal(
                barrier_sem, inc=1,
                device_id=(nbr,), device_id_type=pl.DeviceIdType.MESH,
            )
        pl.semaphore_wait(barrier_sem, 2)

        out_ref[pl.ds(my * CH, CH), :] = partial_chunk(my)
        for s in range(N_DEV - 1):
            send_idx = (my - s) % N_DEV
            rdma = pltpu.make_async_remote_copy(
                src_ref=out_ref.at[pl.ds(send_idx * CH, CH), :],
                dst_ref=recv_rs.at[s],
                send_sem=rs_send.at[s],
                recv_sem=rs_recv.at[s],
                device_id=(right,),
                device_id_type=pl.DeviceIdType.MESH,
            )
            rdma.start()
            c = (my - s - 1) % N_DEV
            part = partial_chunk(c)
            rdma.wait()
            out_ref[pl.ds(c * CH, CH), :] = recv_rs[s] + part


        for t in range(N_DEV - 1):
            send_idx = (my + 1 - t) % N_DEV
            recv_idx = (my - t) % N_DEV
            rdma = pltpu.make_async_remote_copy(
                src_ref=out_ref.at[pl.ds(send_idx * CH, CH), :],
                dst_ref=out_ref.at[pl.ds(send_idx * CH, CH), :],
                send_sem=ag_send.at[t],
                recv_sem=ag_recv.at[t],
                device_id=(right,),
                device_id_type=pl.DeviceIdType.MESH,
            )
            rdma.start()
            rdma.wait()
            del recv_idx

        @functools.partial(
            pl.run_scoped, second_barrier=pltpu.SemaphoreType.REGULAR
        )
        def _(second_barrier):
            for nbr in [left, right]:
                pl.semaphore_signal(
                    second_barrier, inc=1,
                    device_id=(nbr,), device_id_type=pl.DeviceIdType.MESH,
                )
            pl.semaphore_wait(second_barrier, 2)

    return pl.pallas_call(
        body,
        out_shape=jax.ShapeDtypeStruct((M, N), jnp.bfloat16),
        in_specs=[
            pl.BlockSpec(memory_space=pltpu.VMEM),
            pl.BlockSpec(memory_space=pltpu.VMEM),
        ],
        out_specs=pl.BlockSpec(memory_space=pltpu.VMEM),
        scratch_shapes=[
            pltpu.VMEM((N_DEV - 1, CH, N), jnp.bfloat16),
            pltpu.SemaphoreType.DMA((N_DEV - 1,)),
            pltpu.SemaphoreType.DMA((N_DEV - 1,)),
            pltpu.SemaphoreType.DMA((N_DEV - 1,)),
            pltpu.SemaphoreType.DMA((N_DEV - 1,)),
        ],
        compiler_params=pltpu.CompilerParams(collective_id=0),
    )(A16, B16)


# baseline (device time: 372706 ns/iter reference)
import jax
import jax.numpy as jnp
from jax import lax
from jax.experimental import pallas as pl
from jax.experimental.pallas import tpu as pltpu
import functools

N_DEV = 4
M = 3072
N = 3072
CH = M // N_DEV


def kernel(A, B):
    A16 = A.astype(jnp.bfloat16)
    B16 = B.astype(jnp.bfloat16)

    def body(a_ref, b_ref, out_ref, recv_rs, rs_send, rs_recv, ag_send, ag_recv):
        my = lax.axis_index("i")
        left = (my - 1) % N_DEV
        right = (my + 1) % N_DEV

        def partial_chunk(c):
            a = a_ref[pl.ds(c * CH, CH), :]
            return jnp.dot(
                a, b_ref[:, :], preferred_element_type=jnp.float32
            ).astype(jnp.bfloat16)

        barrier_sem = pltpu.get_barrier_semaphore()
        for nbr in [left, right]:
            pl.semaphore_signal(
                barrier_sem, inc=1,
                device_id=(nbr,), device_id_type=pl.DeviceIdType.MESH,
            )
        pl.semaphore_wait(barrier_sem, 2)

        out_ref[pl.ds(my * CH, CH), :] = partial_chunk(my)
        for s in range(N_DEV - 1):
            send_idx = (my - s) % N_DEV
            rdma = pltpu.make_async_remote_copy(
                src_ref=out_ref.at[pl.ds(send_idx * CH, CH), :],
                dst_ref=recv_rs.at[s],
                send_sem=rs_send.at[s],
                recv_sem=rs_recv.at[s],
                device_id=(right,),
                device_id_type=pl.DeviceIdType.MESH,
            )
            rdma.start()
            c = (my - s - 1) % N_DEV
            part = partial_chunk(c)
            rdma.wait()
            out_ref[pl.ds(c * CH, CH), :] = recv_rs[s] + part


        for t in range(N_DEV - 1):
            send_idx = (my + 1 - t) % N_DEV
            recv_idx = (my - t) % N_DEV
            rdma = pltpu.make_async_remote_copy(
                src_ref=out_ref.at[pl.ds(send_idx * CH, CH), :],
                dst_ref=out_ref.at[pl.ds(send_idx * CH, CH), :],
                send_sem=ag_send.at[t],
                recv_sem=ag_recv.at[t],
                device_id=(right,),
                device_id_type=pl.DeviceIdType.MESH,
            )
            rdma.start()
            rdma.wait()
            del recv_idx

        @functools.partial(
            pl.run_scoped, second_barrier=pltpu.SemaphoreType.REGULAR
        )
        def _(second_barrier):
            for nbr in [left, right]:
                pl.semaphore_signal(
                    second_barrier, inc=1,
                    device_id=(nbr,), device_id_type=pl.DeviceIdType.MESH,
                )
            pl.semaphore_wait(second_barrier, 2)

    return pl.pallas_call(
        body,
        out_shape=jax.ShapeDtypeStruct((M, N), jnp.bfloat16),
        in_specs=[
            pl.BlockSpec(memory_space=pltpu.VMEM),
            pl.BlockSpec(memory_space=pltpu.VMEM),
        ],
        out_specs=pl.BlockSpec(memory_space=pltpu.VMEM),
        scratch_shapes=[
            pltpu.VMEM((N_DEV - 1, CH, N), jnp.bfloat16),
            pltpu.SemaphoreType.DMA((N_DEV - 1,)),
            pltpu.SemaphoreType.DMA((N_DEV - 1,)),
            pltpu.SemaphoreType.DMA((N_DEV - 1,)),
            pltpu.SemaphoreType.DMA((N_DEV - 1,)),
        ],
        compiler_params=pltpu.CompilerParams(
            collective_id=0,
            vmem_limit_bytes=64 * 1024 * 1024,
        ),
    )(A16, B16)


# device time: 220908 ns/iter; 1.6872x vs baseline; 1.6872x over previous
import jax
import jax.numpy as jnp
from jax import lax
from jax.experimental import pallas as pl
from jax.experimental.pallas import tpu as pltpu
import functools

N_DEV = 4
M = 3072
N = 3072
CH = M // N_DEV
HN = N // 2


def kernel(A, B):
    A16 = A.astype(jnp.bfloat16)
    B16 = B.astype(jnp.bfloat16)

    def body(a_ref, b_ref, out_ref, recv_r, recv_l,
             rs_r_send, rs_r_recv, rs_l_send, rs_l_recv,
             ag_r_send, ag_r_recv, ag_l_send, ag_l_recv):
        my = lax.axis_index("i")
        left = (my - 1) % N_DEV
        right = (my + 1) % N_DEV

        def partial_half(c, h):
            a = a_ref[pl.ds(c * CH, CH), :]
            b = b_ref[:, pl.ds(h * HN, HN)]
            return jnp.dot(a, b, preferred_element_type=jnp.float32).astype(
                jnp.bfloat16
            )

        barrier_sem = pltpu.get_barrier_semaphore()
        for nbr in [left, right]:
            pl.semaphore_signal(
                barrier_sem, inc=1,
                device_id=(nbr,), device_id_type=pl.DeviceIdType.MESH,
            )
        pl.semaphore_wait(barrier_sem, 2)

        out_ref[pl.ds(my * CH, CH), pl.ds(0, HN)] = partial_half(my, 0)
        out_ref[pl.ds(my * CH, CH), pl.ds(HN, HN)] = partial_half(my, 1)

        for s in range(N_DEV - 1):
            sr = (my - s) % N_DEV
            sl = (my + s) % N_DEV
            rdma_r = pltpu.make_async_remote_copy(
                src_ref=out_ref.at[pl.ds(sr * CH, CH), pl.ds(0, HN)],
                dst_ref=recv_r.at[s],
                send_sem=rs_r_send.at[s], recv_sem=rs_r_recv.at[s],
                device_id=(right,), device_id_type=pl.DeviceIdType.MESH,
            )
            rdma_l = pltpu.make_async_remote_copy(
                src_ref=out_ref.at[pl.ds(sl * CH, CH), pl.ds(HN, HN)],
                dst_ref=recv_l.at[s],
                send_sem=rs_l_send.at[s], recv_sem=rs_l_recv.at[s],
                device_id=(left,), device_id_type=pl.DeviceIdType.MESH,
            )
            rdma_r.start()
            rdma_l.start()
            cr = (my - s - 1) % N_DEV
            cl = (my + s + 1) % N_DEV
            part_r = partial_half(cr, 0)
            part_l = partial_half(cl, 1)
            rdma_r.wait()
            out_ref[pl.ds(cr * CH, CH), pl.ds(0, HN)] = recv_r[s] + part_r
            rdma_l.wait()
            out_ref[pl.ds(cl * CH, CH), pl.ds(HN, HN)] = recv_l[s] + part_l


        for t in range(N_DEV - 1):
            sr = (my + 1 - t) % N_DEV
            sl = (my - 1 + t) % N_DEV
            rdma_r = pltpu.make_async_remote_copy(
                src_ref=out_ref.at[pl.ds(sr * CH, CH), pl.ds(0, HN)],
                dst_ref=out_ref.at[pl.ds(sr * CH, CH), pl.ds(0, HN)],
                send_sem=ag_r_send.at[t], recv_sem=ag_r_recv.at[t],
                device_id=(right,), device_id_type=pl.DeviceIdType.MESH,
            )
            rdma_l = pltpu.make_async_remote_copy(
                src_ref=out_ref.at[pl.ds(sl * CH, CH), pl.ds(HN, HN)],
                dst_ref=out_ref.at[pl.ds(sl * CH, CH), pl.ds(HN, HN)],
                send_sem=ag_l_send.at[t], recv_sem=ag_l_recv.at[t],
                device_id=(left,), device_id_type=pl.DeviceIdType.MESH,
            )
            rdma_r.start()
            rdma_l.start()
            rdma_r.wait()
            rdma_l.wait()

        @functools.partial(
            pl.run_scoped, second_barrier=pltpu.SemaphoreType.REGULAR
        )
        def _(second_barrier):
            for nbr in [left, right]:
                pl.semaphore_signal(
                    second_barrier, inc=1,
                    device_id=(nbr,), device_id_type=pl.DeviceIdType.MESH,
                )
            pl.semaphore_wait(second_barrier, 2)

    sem3 = pltpu.SemaphoreType.DMA((N_DEV - 1,))
    return pl.pallas_call(
        body,
        out_shape=jax.ShapeDtypeStruct((M, N), jnp.bfloat16),
        in_specs=[
            pl.BlockSpec(memory_space=pltpu.VMEM),
            pl.BlockSpec(memory_space=pltpu.VMEM),
        ],
        out_specs=pl.BlockSpec(memory_space=pltpu.VMEM),
        scratch_shapes=[
            pltpu.VMEM((N_DEV - 1, CH, HN), jnp.bfloat16),
            pltpu.VMEM((N_DEV - 1, CH, HN), jnp.bfloat16),
            sem3, sem3,
            sem3, sem3,
            sem3, sem3,
            sem3, sem3,
        ],
        compiler_params=pltpu.CompilerParams(
            collective_id=0,
            vmem_limit_bytes=64 * 1024 * 1024,
        ),
    )(A16, B16)


# device time: 219960 ns/iter; 1.6944x vs baseline; 1.0043x over previous
import jax
import jax.numpy as jnp
from jax import lax
from jax.experimental import pallas as pl
from jax.experimental.pallas import tpu as pltpu
import functools

N_DEV = 4
M = 3072
N = 3072
CH = M // N_DEV
HN = N // 2


def kernel(A, B):
    A16 = A.astype(jnp.bfloat16)
    B16 = B.astype(jnp.bfloat16)

    def body(a_ref, b_ref, out_ref, recv_r, recv_l,
             rs_r_send, rs_r_recv, rs_l_send, rs_l_recv,
             ag_r_send, ag_r_recv, ag_l_send, ag_l_recv):
        my = lax.axis_index("i")
        left = (my - 1) % N_DEV
        right = (my + 1) % N_DEV

        def partial_half(c, h):
            a = a_ref[pl.ds(c * CH, CH), :]
            b = b_ref[:, pl.ds(h * HN, HN)]
            return jnp.dot(a, b, preferred_element_type=jnp.float32).astype(
                jnp.bfloat16
            )

        barrier_sem = pltpu.get_barrier_semaphore()
        for nbr in [left, right]:
            pl.semaphore_signal(
                barrier_sem, inc=1,
                device_id=(nbr,), device_id_type=pl.DeviceIdType.MESH,
            )
        pl.semaphore_wait(barrier_sem, 2)

        def mk_rs_r(s):
            sr = (my - s) % N_DEV
            return pltpu.make_async_remote_copy(
                src_ref=out_ref.at[pl.ds(sr * CH, CH), pl.ds(0, HN)],
                dst_ref=recv_r.at[s],
                send_sem=rs_r_send.at[s], recv_sem=rs_r_recv.at[s],
                device_id=(right,), device_id_type=pl.DeviceIdType.MESH,
            )

        def mk_rs_l(s):
            sl = (my + s) % N_DEV
            return pltpu.make_async_remote_copy(
                src_ref=out_ref.at[pl.ds(sl * CH, CH), pl.ds(HN, HN)],
                dst_ref=recv_l.at[s],
                send_sem=rs_l_send.at[s], recv_sem=rs_l_recv.at[s],
                device_id=(left,), device_id_type=pl.DeviceIdType.MESH,
            )

        def mk_ag_r(t):
            sr = (my + 1 - t) % N_DEV
            return pltpu.make_async_remote_copy(
                src_ref=out_ref.at[pl.ds(sr * CH, CH), pl.ds(0, HN)],
                dst_ref=out_ref.at[pl.ds(sr * CH, CH), pl.ds(0, HN)],
                send_sem=ag_r_send.at[t], recv_sem=ag_r_recv.at[t],
                device_id=(right,), device_id_type=pl.DeviceIdType.MESH,
            )

        def mk_ag_l(t):
            sl = (my - 1 + t) % N_DEV
            return pltpu.make_async_remote_copy(
                src_ref=out_ref.at[pl.ds(sl * CH, CH), pl.ds(HN, HN)],
                dst_ref=out_ref.at[pl.ds(sl * CH, CH), pl.ds(HN, HN)],
                send_sem=ag_l_send.at[t], recv_sem=ag_l_recv.at[t],
                device_id=(left,), device_id_type=pl.DeviceIdType.MESH,
            )

        out_ref[pl.ds(my * CH, CH), pl.ds(0, HN)] = partial_half(my, 0)
        r = mk_rs_r(0)
        r.start()
        out_ref[pl.ds(my * CH, CH), pl.ds(HN, HN)] = partial_half(my, 1)
        l = mk_rs_l(0)
        l.start()
        part_r = partial_half((my - 1) % N_DEV, 0)
        part_l = partial_half((my + 1) % N_DEV, 1)

        for s in range(N_DEV - 1):
            cr = (my - s - 1) % N_DEV
            cl = (my + s + 1) % N_DEV
            r.wait()
            out_ref[pl.ds(cr * CH, CH), pl.ds(0, HN)] = recv_r[s] + part_r
            r = mk_rs_r(s + 1) if s < N_DEV - 2 else mk_ag_r(0)
            r.start()
            l.wait()
            out_ref[pl.ds(cl * CH, CH), pl.ds(HN, HN)] = recv_l[s] + part_l
            l = mk_rs_l(s + 1) if s < N_DEV - 2 else mk_ag_l(0)
            l.start()
            if s < N_DEV - 2:
                part_r = partial_half((my - s - 2) % N_DEV, 0)
                part_l = partial_half((my + s + 2) % N_DEV, 1)

        for t in range(N_DEV - 1):
            r.wait()
            l.wait()
            if t < N_DEV - 2:
                r = mk_ag_r(t + 1)
                r.start()
                l = mk_ag_l(t + 1)
                l.start()

        @functools.partial(
            pl.run_scoped, second_barrier=pltpu.SemaphoreType.REGULAR
        )
        def _(second_barrier):
            for nbr in [left, right]:
                pl.semaphore_signal(
                    second_barrier, inc=1,
                    device_id=(nbr,), device_id_type=pl.DeviceIdType.MESH,
                )
            pl.semaphore_wait(second_barrier, 2)

    sem3 = pltpu.SemaphoreType.DMA((N_DEV - 1,))
    return pl.pallas_call(
        body,
        out_shape=jax.ShapeDtypeStruct((M, N), jnp.bfloat16),
        in_specs=[
            pl.BlockSpec(memory_space=pltpu.VMEM),
            pl.BlockSpec(memory_space=pltpu.VMEM),
        ],
        out_specs=pl.BlockSpec(memory_space=pltpu.VMEM),
        scratch_shapes=[
            pltpu.VMEM((N_DEV - 1, CH, HN), jnp.bfloat16),
            pltpu.VMEM((N_DEV - 1, CH, HN), jnp.bfloat16),
            sem3, sem3,
            sem3, sem3,
            sem3, sem3,
            sem3, sem3,
        ],
        compiler_params=pltpu.CompilerParams(
            collective_id=0,
            vmem_limit_bytes=64 * 1024 * 1024,
        ),
    )(A16, B16)


# device time: 210385 ns/iter; 1.7715x vs baseline; 1.0455x over previous
import jax
import jax.numpy as jnp
from jax import lax
from jax.experimental import pallas as pl
from jax.experimental.pallas import tpu as pltpu
import functools

N_DEV = 4
M = 3072
N = 3072
CH = M // N_DEV
HN = N // 2
SB = HN // 2
NSUB = 2


def kernel(A, B):
    A16 = A.astype(jnp.bfloat16)
    B16 = B.astype(jnp.bfloat16)

    def body(a_ref, b_ref, out_ref, recv_r, recv_l,
             rs_r_send, rs_r_recv, rs_l_send, rs_l_recv,
             ag_r_send, ag_r_recv, ag_l_send, ag_l_recv):
        my = lax.axis_index("i")
        left = (my - 1) % N_DEV
        right = (my + 1) % N_DEV

        def partial_half(c, h):
            a = a_ref[pl.ds(c * CH, CH), :]
            b = b_ref[:, pl.ds(h * HN, HN)]
            return jnp.dot(a, b, preferred_element_type=jnp.float32).astype(
                jnp.bfloat16
            )

        barrier_sem = pltpu.get_barrier_semaphore()
        for nbr in [left, right]:
            pl.semaphore_signal(
                barrier_sem, inc=1,
                device_id=(nbr,), device_id_type=pl.DeviceIdType.MESH,
            )
        pl.semaphore_wait(barrier_sem, 2)

        def mk_rs_r(s, u):
            sr = (my - s) % N_DEV
            return pltpu.make_async_remote_copy(
                src_ref=out_ref.at[pl.ds(sr * CH, CH), pl.ds(u * SB, SB)],
                dst_ref=recv_r.at[s, u],
                send_sem=rs_r_send.at[s, u], recv_sem=rs_r_recv.at[s, u],
                device_id=(right,), device_id_type=pl.DeviceIdType.MESH,
            )

        def mk_rs_l(s, u):
            sl = (my + s) % N_DEV
            return pltpu.make_async_remote_copy(
                src_ref=out_ref.at[pl.ds(sl * CH, CH), pl.ds(HN + u * SB, SB)],
                dst_ref=recv_l.at[s, u],
                send_sem=rs_l_send.at[s, u], recv_sem=rs_l_recv.at[s, u],
                device_id=(left,), device_id_type=pl.DeviceIdType.MESH,
            )

        def mk_ag_r(t, u):
            sr = (my + 1 - t) % N_DEV
            sl_ref = out_ref.at[pl.ds(sr * CH, CH), pl.ds(u * SB, SB)]
            return pltpu.make_async_remote_copy(
                src_ref=sl_ref, dst_ref=sl_ref,
                send_sem=ag_r_send.at[t, u], recv_sem=ag_r_recv.at[t, u],
                device_id=(right,), device_id_type=pl.DeviceIdType.MESH,
            )

        def mk_ag_l(t, u):
            sl = (my - 1 + t) % N_DEV
            sl_ref = out_ref.at[pl.ds(sl * CH, CH), pl.ds(HN + u * SB, SB)]
            return pltpu.make_async_remote_copy(
                src_ref=sl_ref, dst_ref=sl_ref,
                send_sem=ag_l_send.at[t, u], recv_sem=ag_l_recv.at[t, u],
                device_id=(left,), device_id_type=pl.DeviceIdType.MESH,
            )

        out_ref[pl.ds(my * CH, CH), pl.ds(0, HN)] = partial_half(my, 0)
        r = [mk_rs_r(0, u) for u in range(NSUB)]
        for u in range(NSUB):
            r[u].start()
        out_ref[pl.ds(my * CH, CH), pl.ds(HN, HN)] = partial_half(my, 1)
        l = [mk_rs_l(0, u) for u in range(NSUB)]
        for u in range(NSUB):
            l[u].start()
        part_r = partial_half((my - 1) % N_DEV, 0)
        part_l = partial_half((my + 1) % N_DEV, 1)

        for s in range(N_DEV - 1):
            cr = (my - s - 1) % N_DEV
            cl = (my + s + 1) % N_DEV
            nxt_r, nxt_l = [None] * NSUB, [None] * NSUB
            for u in range(NSUB):
                r[u].wait()
                out_ref[pl.ds(cr * CH, CH), pl.ds(u * SB, SB)] = (
                    recv_r[s, u] + part_r[:, u * SB:(u + 1) * SB]
                )
                nxt_r[u] = mk_rs_r(s + 1, u) if s < N_DEV - 2 else mk_ag_r(0, u)
                nxt_r[u].start()
                l[u].wait()
                out_ref[pl.ds(cl * CH, CH), pl.ds(HN + u * SB, SB)] = (
                    recv_l[s, u] + part_l[:, u * SB:(u + 1) * SB]
                )
                nxt_l[u] = mk_rs_l(s + 1, u) if s < N_DEV - 2 else mk_ag_l(0, u)
                nxt_l[u].start()
            r, l = nxt_r, nxt_l
            if s < N_DEV - 2:
                part_r = partial_half((my - s - 2) % N_DEV, 0)
                part_l = partial_half((my + s + 2) % N_DEV, 1)

        for t in range(N_DEV - 1):
            nxt_r, nxt_l = [None] * NSUB, [None] * NSUB
            for u in range(NSUB):
                r[u].wait()
                l[u].wait()
                if t < N_DEV - 2:
                    nxt_r[u] = mk_ag_r(t + 1, u)
                    nxt_r[u].start()
                    nxt_l[u] = mk_ag_l(t + 1, u)
                    nxt_l[u].start()
            r, l = nxt_r, nxt_l

        @functools.partial(
            pl.run_scoped, second_barrier=pltpu.SemaphoreType.REGULAR
        )
        def _(second_barrier):
            for nbr in [left, right]:
                pl.semaphore_signal(
                    second_barrier, inc=1,
                    device_id=(nbr,), device_id_type=pl.DeviceIdType.MESH,
                )
            pl.semaphore_wait(second_barrier, 2)

    sems = pltpu.SemaphoreType.DMA((N_DEV - 1, NSUB))
    return pl.pallas_call(
        body,
        out_shape=jax.ShapeDtypeStruct((M, N), jnp.bfloat16),
        in_specs=[
            pl.BlockSpec(memory_space=pltpu.VMEM),
            pl.BlockSpec(memory_space=pltpu.VMEM),
        ],
        out_specs=pl.BlockSpec(memory_space=pltpu.VMEM),
        scratch_shapes=[
            pltpu.VMEM((N_DEV - 1, NSUB, CH, SB), jnp.bfloat16),
            pltpu.VMEM((N_DEV - 1, NSUB, CH, SB), jnp.bfloat16),
            sems, sems,
            sems, sems,
            sems, sems,
            sems, sems,
        ],
        compiler_params=pltpu.CompilerParams(
            collective_id=0,
            vmem_limit_bytes=64 * 1024 * 1024,
        ),
    )(A16, B16)
